# Initial kernel scaffold; baseline (speedup 1.0000x reference)
import jax
import jax.numpy as jnp
from jax.experimental import pallas as pl


def kernel(x, edge_index, W1, b1, W2, b2):
    raise NotImplementedError("write your pallas kernel here")


# speedup vs baseline: 13.0979x; 13.0979x over previous
import functools

import jax
import jax.numpy as jnp
from jax import lax
from jax.experimental import pallas as pl
from jax.experimental.pallas import tpu as pltpu
from jax.experimental.pallas import tpu_sc as plsc

_N, _E, _D = 10000, 320000, 128
_NC, _NS = 2, 16
_NW = _NC * _NS
_EPW = _E // _NW
_C = 80
_NCH = _EPW // _C
_RPT = 640
_NPAD = 10240
_DPT = _NPAD // _NS

_mesh = plsc.VectorSubcoreMesh(core_axis_name="c", subcore_axis_name="s")


@functools.partial(
    pl.kernel,
    mesh=_mesh,
    out_type=jax.ShapeDtypeStruct((2, _NPAD), jnp.float32),
    scratch_types=[
        pltpu.VMEM((_C,), jnp.int32),
        pltpu.VMEM((_C,), jnp.float32),
        pltpu.VMEM((_DPT,), jnp.float32),
        pltpu.VMEM_SHARED((_NPAD,), jnp.float32),
    ],
)
def _sc_degree(dst_hbm, out_hbm, idx_v, ones_v, zeros_v, acc_sh):
    c = lax.axis_index("c")
    s = lax.axis_index("s")
    wid = s * _NC + c
    for i in range(_C // 16):
        ones_v[pl.ds(i * 16, 16)] = jnp.ones((16,), jnp.float32)
    for i in range(_DPT // 16):
        zeros_v[pl.ds(i * 16, 16)] = jnp.zeros((16,), jnp.float32)
    pltpu.sync_copy(zeros_v, acc_sh.at[pl.ds(s * _DPT, _DPT)])
    plsc.subcore_barrier()

    def body(j, carry):
        off = wid * _EPW + j * _C
        pltpu.sync_copy(dst_hbm.at[pl.ds(off, _C)], idx_v)
        pltpu.sync_copy(ones_v, acc_sh.at[idx_v], add=True)
        return carry

    lax.fori_loop(0, _NCH, body, 0)
    plsc.subcore_barrier()
    pltpu.sync_copy(acc_sh.at[pl.ds(s * _DPT, _DPT)],
                    out_hbm.at[c, pl.ds(s * _DPT, _DPT)])


@functools.partial(
    pl.kernel,
    mesh=_mesh,
    out_type=jax.ShapeDtypeStruct((2, _N, _D), jnp.float32),
    scratch_types=[
        pltpu.VMEM((_C,), jnp.int32),
        pltpu.VMEM((_C,), jnp.int32),
        pltpu.VMEM((_C, _D), jnp.float32),
        pltpu.VMEM_SHARED((_N, _D), jnp.float32),
        pltpu.SemaphoreType.DMA,
    ],
)
def _sc_agg(y_hbm, src_hbm, dst_hbm, out_hbm, src_v, dst_v, rows_v, acc_sh, sem):
    c = lax.axis_index("c")
    s = lax.axis_index("s")
    wid = s * _NC + c
    row0 = pl.multiple_of(jnp.minimum(s * _RPT, _N - _RPT), 8)
    pltpu.sync_copy(y_hbm.at[pl.ds(row0, _RPT)],
                    acc_sh.at[pl.ds(row0, _RPT)])
    plsc.subcore_barrier()

    def body(j, carry):
        off = wid * _EPW + j * _C
        pltpu.sync_copy(src_hbm.at[pl.ds(off, _C)], src_v)
        pltpu.sync_copy(dst_hbm.at[pl.ds(off, _C)], dst_v)
        pltpu.async_copy(y_hbm.at[src_v], rows_v, sem).wait()
        pltpu.sync_copy(rows_v, acc_sh.at[dst_v], add=True)
        return carry

    lax.fori_loop(0, _NCH, body, 0)
    plsc.subcore_barrier()
    pltpu.sync_copy(acc_sh.at[pl.ds(row0, _RPT)],
                    out_hbm.at[c, pl.ds(row0, _RPT)])


_R = 400
_G = _N // _R


def _d1_body(x_ref, w_ref, dinv_ref, y_ref):
    y_ref[...] = jnp.dot(x_ref[...], w_ref[...],
                         preferred_element_type=jnp.float32) * dinv_ref[...]


_dense1 = pl.pallas_call(
    _d1_body,
    grid=(_G,),
    in_specs=[
        pl.BlockSpec((_R, _D), lambda i: (i, 0)),
        pl.BlockSpec((_D, _D), lambda i: (0, 0)),
        pl.BlockSpec((_R, 1), lambda i: (i, 0)),
    ],
    out_specs=pl.BlockSpec((_R, _D), lambda i: (i, 0)),
    out_shape=jax.ShapeDtypeStruct((_N, _D), jnp.float32),
)


def _d2_body(p0_ref, p1_ref, y1_ref, dinv_ref, b1_ref, w2_ref, y2_ref):
    agg = p0_ref[0] + p1_ref[0] - y1_ref[...]
    h1 = jnp.maximum(agg * dinv_ref[...] + b1_ref[...], 0.0)
    y2_ref[...] = jnp.dot(h1, w2_ref[...],
                          preferred_element_type=jnp.float32) * dinv_ref[...]


_dense2 = pl.pallas_call(
    _d2_body,
    grid=(_G,),
    in_specs=[
        pl.BlockSpec((1, _R, _D), lambda i: (0, i, 0)),
        pl.BlockSpec((1, _R, _D), lambda i: (1, i, 0)),
        pl.BlockSpec((_R, _D), lambda i: (i, 0)),
        pl.BlockSpec((_R, 1), lambda i: (i, 0)),
        pl.BlockSpec((1, _D), lambda i: (0, 0)),
        pl.BlockSpec((_D, _D), lambda i: (0, 0)),
    ],
    out_specs=pl.BlockSpec((_R, _D), lambda i: (i, 0)),
    out_shape=jax.ShapeDtypeStruct((_N, _D), jnp.float32),
)


def _d3_body(q0_ref, q1_ref, y2_ref, dinv_ref, b2_ref, out_ref):
    h = (q0_ref[0] + q1_ref[0] - y2_ref[...]) * dinv_ref[...] + b2_ref[...]
    m = jnp.max(h, axis=1, keepdims=True)
    hm = h - m
    out_ref[...] = hm - jnp.log(jnp.sum(jnp.exp(hm), axis=1, keepdims=True))


_final = pl.pallas_call(
    _d3_body,
    grid=(_G,),
    in_specs=[
        pl.BlockSpec((1, _R, _D), lambda i: (0, i, 0)),
        pl.BlockSpec((1, _R, _D), lambda i: (1, i, 0)),
        pl.BlockSpec((_R, _D), lambda i: (i, 0)),
        pl.BlockSpec((_R, 1), lambda i: (i, 0)),
        pl.BlockSpec((1, _D), lambda i: (0, 0)),
    ],
    out_specs=pl.BlockSpec((_R, _D), lambda i: (i, 0)),
    out_shape=jax.ShapeDtypeStruct((_N, _D), jnp.float32),
)


def kernel(x, edge_index, W1, b1, W2, b2):
    src = edge_index[0]
    dst = edge_index[1]
    degp = _sc_degree(dst)
    deg = degp[0, :_N] + degp[1, :_N] + 1.0
    dinv = lax.rsqrt(deg)[:, None]
    y1 = _dense1(x, W1, dinv)
    p = _sc_agg(y1, src, dst)
    y2 = _dense2(p, p, y1, dinv, b1[None, :], W2)
    q = _sc_agg(y2, src, dst)
    return _final(q, q, y2, dinv, b2[None, :])


# speedup vs baseline: 28.8401x; 2.2019x over previous
import functools

import jax
import jax.numpy as jnp
from jax import lax
from jax.experimental import pallas as pl
from jax.experimental.pallas import tpu as pltpu
from jax.experimental.pallas import tpu_sc as plsc

_N, _E, _D = 10000, 320000, 128
_NC, _NS = 2, 16
_NW = _NC * _NS
_EPW = _E // _NW
_C = 80
_NCH = _EPW // _C
_RPT = 640
_NPAD = 10240
_DPT = _NPAD // _NS

_mesh = plsc.VectorSubcoreMesh(core_axis_name="c", subcore_axis_name="s")


@functools.partial(
    pl.kernel,
    mesh=_mesh,
    out_type=jax.ShapeDtypeStruct((2, _NPAD), jnp.float32),
    scratch_types=[
        pltpu.VMEM((_NCH, _C), jnp.int32),
        pltpu.VMEM((_C,), jnp.float32),
        pltpu.VMEM((_DPT,), jnp.float32),
        pltpu.VMEM_SHARED((_NPAD,), jnp.float32),
    ],
)
def _sc_degree(dst_hbm, out_hbm, dsts_v, ones_v, zeros_v, acc_sh):
    c = lax.axis_index("c")
    s = lax.axis_index("s")
    wid = s * _NC + c
    for i in range(_C // 16):
        ones_v[pl.ds(i * 16, 16)] = jnp.ones((16,), jnp.float32)
    for i in range(_DPT // 16):
        zeros_v[pl.ds(i * 16, 16)] = jnp.zeros((16,), jnp.float32)
    pltpu.sync_copy(dst_hbm.at[wid], dsts_v)
    pltpu.sync_copy(zeros_v, acc_sh.at[pl.ds(s * _DPT, _DPT)])
    plsc.subcore_barrier()

    def body(j, carry):
        pltpu.sync_copy(ones_v, acc_sh.at[dsts_v.at[j]], add=True)
        return carry

    lax.fori_loop(0, _NCH, body, 0)
    plsc.subcore_barrier()
    pltpu.sync_copy(acc_sh.at[pl.ds(s * _DPT, _DPT)],
                    out_hbm.at[c, pl.ds(s * _DPT, _DPT)])


@functools.partial(
    pl.kernel,
    mesh=_mesh,
    out_type=jax.ShapeDtypeStruct((2, _N, _D), jnp.float32),
    scratch_types=[
        pltpu.VMEM((64, _C), jnp.int32),
        pltpu.VMEM((64, _C), jnp.int32),
        pltpu.VMEM((_C, _D), jnp.float32),
        pltpu.VMEM((_C, _D), jnp.float32),
        pltpu.VMEM_SHARED((_N, _D), jnp.float32),
        pltpu.SemaphoreType.DMA,
        pltpu.SemaphoreType.DMA,
    ],
)
def _sc_agg(y_hbm, src_hbm, dst_hbm, out_hbm,
            srcs_v, dsts_v, rows_a, rows_b, acc_sh, sem_a, sem_b):
    c = lax.axis_index("c")
    s = lax.axis_index("s")
    wid = s * _NC + c
    row0 = pl.multiple_of(jnp.minimum(s * _RPT, _N - _RPT), 8)
    pltpu.sync_copy(y_hbm.at[pl.ds(row0, _RPT)],
                    acc_sh.at[pl.ds(row0, _RPT)])
    plsc.subcore_barrier()

    def _gather(j, buf, sem):
        return pltpu.async_copy(y_hbm.at[srcs_v.at[j]], buf, sem)

    def _wait(j, buf, sem):
        pltpu.make_async_copy(y_hbm.at[srcs_v.at[j]], buf, sem).wait()

    def _scatter(j, buf):
        pltpu.sync_copy(buf, acc_sh.at[dsts_v.at[j]], add=True)

    def _load_idx(start, k):
        pltpu.sync_copy(src_hbm.at[wid, pl.ds(start, k)],
                        srcs_v.at[pl.ds(0, k)])
        pltpu.sync_copy(dst_hbm.at[wid, pl.ds(start, k)],
                        dsts_v.at[pl.ds(0, k)])

    def _run_pass(k):
        _gather(0, rows_a, sem_a)

        def pair(jj, carry):
            j0 = 2 * jj
            _gather(j0 + 1, rows_b, sem_b)
            _wait(j0, rows_a, sem_a)
            _scatter(j0, rows_a)
            _gather(j0 + 2, rows_a, sem_a)
            _wait(j0 + 1, rows_b, sem_b)
            _scatter(j0 + 1, rows_b)
            return carry

        lax.fori_loop(0, (k - 1) // 2, pair, 0)
        if k % 2 == 1:
            _wait(k - 1, rows_a, sem_a)
            _scatter(k - 1, rows_a)
        else:
            _gather(k - 1, rows_b, sem_b)
            _wait(k - 2, rows_a, sem_a)
            _scatter(k - 2, rows_a)
            _wait(k - 1, rows_b, sem_b)
            _scatter(k - 1, rows_b)

    _load_idx(0, 64)
    _run_pass(64)
    _load_idx(64, _NCH - 64)
    _run_pass(_NCH - 64)

    plsc.subcore_barrier()
    pltpu.sync_copy(acc_sh.at[pl.ds(row0, _RPT)],
                    out_hbm.at[c, pl.ds(row0, _RPT)])


_R = 400
_G = _N // _R


def _d1_body(x_ref, w_ref, dinv_ref, y_ref):
    y_ref[...] = jnp.dot(x_ref[...], w_ref[...],
                         preferred_element_type=jnp.float32) * dinv_ref[...]


_dense1 = pl.pallas_call(
    _d1_body,
    grid=(_G,),
    in_specs=[
        pl.BlockSpec((_R, _D), lambda i: (i, 0)),
        pl.BlockSpec((_D, _D), lambda i: (0, 0)),
        pl.BlockSpec((_R, 1), lambda i: (i, 0)),
    ],
    out_specs=pl.BlockSpec((_R, _D), lambda i: (i, 0)),
    out_shape=jax.ShapeDtypeStruct((_N, _D), jnp.float32),
)


def _d2_body(p0_ref, p1_ref, y1_ref, dinv_ref, b1_ref, w2_ref, y2_ref):
    agg = p0_ref[0] + p1_ref[0] - y1_ref[...]
    h1 = jnp.maximum(agg * dinv_ref[...] + b1_ref[...], 0.0)
    y2_ref[...] = jnp.dot(h1, w2_ref[...],
                          preferred_element_type=jnp.float32) * dinv_ref[...]


_dense2 = pl.pallas_call(
    _d2_body,
    grid=(_G,),
    in_specs=[
        pl.BlockSpec((1, _R, _D), lambda i: (0, i, 0)),
        pl.BlockSpec((1, _R, _D), lambda i: (1, i, 0)),
        pl.BlockSpec((_R, _D), lambda i: (i, 0)),
        pl.BlockSpec((_R, 1), lambda i: (i, 0)),
        pl.BlockSpec((1, _D), lambda i: (0, 0)),
        pl.BlockSpec((_D, _D), lambda i: (0, 0)),
    ],
    out_specs=pl.BlockSpec((_R, _D), lambda i: (i, 0)),
    out_shape=jax.ShapeDtypeStruct((_N, _D), jnp.float32),
)


def _d3_body(q0_ref, q1_ref, y2_ref, dinv_ref, b2_ref, out_ref):
    h = (q0_ref[0] + q1_ref[0] - y2_ref[...]) * dinv_ref[...] + b2_ref[...]
    m = jnp.max(h, axis=1, keepdims=True)
    hm = h - m
    out_ref[...] = hm - jnp.log(jnp.sum(jnp.exp(hm), axis=1, keepdims=True))


_final = pl.pallas_call(
    _d3_body,
    grid=(_G,),
    in_specs=[
        pl.BlockSpec((1, _R, _D), lambda i: (0, i, 0)),
        pl.BlockSpec((1, _R, _D), lambda i: (1, i, 0)),
        pl.BlockSpec((_R, _D), lambda i: (i, 0)),
        pl.BlockSpec((_R, 1), lambda i: (i, 0)),
        pl.BlockSpec((1, _D), lambda i: (0, 0)),
    ],
    out_specs=pl.BlockSpec((_R, _D), lambda i: (i, 0)),
    out_shape=jax.ShapeDtypeStruct((_N, _D), jnp.float32),
)


def kernel(x, edge_index, W1, b1, W2, b2):
    src = edge_index[0].reshape(_NW, _NCH, _C)
    dst = edge_index[1].reshape(_NW, _NCH, _C)
    degp = _sc_degree(dst)
    deg = degp[0, :_N] + degp[1, :_N] + 1.0
    dinv = lax.rsqrt(deg)[:, None]
    y1 = _dense1(x, W1, dinv)
    p = _sc_agg(y1, src, dst)
    y2 = _dense2(p, p, y1, dinv, b1[None, :], W2)
    q = _sc_agg(y2, src, dst)
    return _final(q, q, y2, dinv, b2[None, :])
